# baseline (speedup 1.0000x reference)
import functools
import jax
import jax.numpy as jnp
from jax import lax
from jax.experimental import pallas as pl
from jax.experimental.pallas import tpu as pltpu
from jax.experimental.pallas import tpu_sc as plsc

_Q, _K, _D, _OUT = 16, 100000, 128, 18
_MIN_DIST = float(_OUT)

_BLK = 2000
_KTC = 54000
_NBLK = _KTC // _BLK
_KSC = _K - _KTC
_NW = 32
_RPW = _KSC // _NW
_NG = (_RPW // 16 + 3) // 2 * 2


def _sc_body(dspl_hbm, daspl_hbm, obsT_hbm, outv_hbm, outi_hbm,
             dtab, datab, buf0, buf1, minbuf, idxbuf,
             sem0, sem1, semt0, semt1):
    c = lax.axis_index("c")
    s = lax.axis_index("s")
    w = s * 2 + c
    a0 = (w * _RPW) // 16 * 16
    a1 = jnp.where(w == _NW - 1, _KSC, ((w + 1) * _RPW) // 16 * 16)
    last = a1 - 16

    cpt0 = pltpu.make_async_copy(dspl_hbm, dtab, semt0)
    cpt1 = pltpu.make_async_copy(daspl_hbm, datab, semt1)
    cpt0.start()
    cpt1.start()

    for q in range(_Q):
        minbuf[pl.ds(q * 16, 16)] = jnp.full((16,), jnp.inf, jnp.float32)
        idxbuf[pl.ds(q * 16, 16)] = jnp.zeros((16,), jnp.int32)

    lane16 = lax.broadcasted_iota(jnp.int32, (16,), 0)

    def gstart(g):
        return jnp.minimum(a0 + 16 * g, last)

    def dma(g, buf, sem):
        return pltpu.make_async_copy(
            obsT_hbm.at[:, pl.ds(gstart(g), 16)], buf, sem)

    dma(0, buf0, sem0).start()
    dma(1, buf1, sem1).start()
    cpt0.wait()
    cpt1.wait()

    def group(g, buf, sem):
        dma(g, buf, sem).wait()
        rowvec = lane16 + (gstart(g) + _KTC)

        def dstep(d, accs):
            ovec = buf[d]
            oabs = jnp.abs(ovec)
            out = []
            for q in range(_Q):
                dq = dtab[_D * q + d]
                daq = datab[_D * q + d]
                num = jnp.abs(ovec - dq)
                den = jnp.maximum(oabs + daq, 1e-30)
                out.append(accs[q] + num / den)
            return tuple(out)

        zero = jnp.zeros((16,), jnp.float32)
        accs = lax.fori_loop(0, _D, dstep, tuple(zero for _ in range(_Q)))

        for q in range(_Q):
            sl = pl.ds(q * 16, 16)
            mv = minbuf[sl]
            better = accs[q] < mv
            minbuf[sl] = jnp.where(better, accs[q], mv)
            idxbuf[sl] = jnp.where(better, rowvec, idxbuf[sl])

        @pl.when(g + 2 < _NG)
        def _pref():
            dma(g + 2, buf, sem).start()

    def outer(o, carry):
        group(o * 2, buf0, sem0)
        group(o * 2 + 1, buf1, sem1)
        return carry

    lax.fori_loop(0, _NG // 2, outer, jnp.int32(0))

    pltpu.sync_copy(minbuf, outv_hbm.at[w])
    pltpu.sync_copy(idxbuf, outi_hbm.at[w])


def _sc_scan(dsplat, dasplat, obsT):
    mesh = plsc.VectorSubcoreMesh(core_axis_name="c", subcore_axis_name="s")
    f = pl.kernel(
        _sc_body,
        out_type=[jax.ShapeDtypeStruct((_NW, _Q * 16), jnp.float32),
                  jax.ShapeDtypeStruct((_NW, _Q * 16), jnp.int32)],
        mesh=mesh,
        scratch_types=[
            pltpu.VMEM((_Q * _D, 16), jnp.float32),
            pltpu.VMEM((_Q * _D, 16), jnp.float32),
            pltpu.VMEM((_D, 16), jnp.float32),
            pltpu.VMEM((_D, 16), jnp.float32),
            pltpu.VMEM((_Q * 16,), jnp.float32),
            pltpu.VMEM((_Q * 16,), jnp.int32),
            pltpu.SemaphoreType.DMA,
            pltpu.SemaphoreType.DMA,
            pltpu.SemaphoreType.DMA,
            pltpu.SemaphoreType.DMA,
        ],
        compiler_params=pltpu.CompilerParams(use_tc_tiling_on_sc=False),
    )
    return f(dsplat, dasplat, obsT)


def _tc_body(x_ref, obs_ref, outv_ref, outi_ref):
    b = pl.program_id(0)

    @pl.when(b == 0)
    def _init():
        for q in range(_Q):
            outv_ref[q] = jnp.inf
            outi_ref[q] = 0

    x = x_ref[...]
    deltas = jnp.concatenate([x[:1], x[:-1] - x[1:]], axis=0)
    dabs = jnp.abs(deltas)

    obs = obs_ref[...]
    oabs = jnp.abs(obs)
    base = b * _BLK
    ones = jnp.ones((_D, 8), jnp.float32)

    for q in range(_Q):
        num = jnp.abs(obs - deltas[q][None, :])
        den = jnp.maximum(oabs + dabs[q][None, :], 1e-30)
        term = num / den
        dist8 = jax.lax.dot_general(
            term, ones, (((1,), (0,)), ((), ())),
            preferred_element_type=jnp.float32,
            precision=jax.lax.Precision.HIGHEST,
        )
        bmin = jnp.min(dist8)

        @pl.when(bmin < outv_ref[q])
        def _upd():
            rows = jax.lax.broadcasted_iota(jnp.int32, (_BLK, 8), 0) + base
            bidx = jnp.min(
                jnp.where(dist8 == bmin, rows, jnp.int32(2**31 - 1))
            )
            outv_ref[q] = bmin
            outi_ref[q] = bidx


def _tc_scan(x, observations):
    grid_spec = pltpu.PrefetchScalarGridSpec(
        num_scalar_prefetch=0,
        grid=(_NBLK,),
        in_specs=[
            pl.BlockSpec((_Q, _D), lambda b: (0, 0)),
            pl.BlockSpec((_BLK, _D), lambda b: (b, 0)),
        ],
        out_specs=[
            pl.BlockSpec(memory_space=pltpu.SMEM),
            pl.BlockSpec(memory_space=pltpu.SMEM),
        ],
    )
    return pl.pallas_call(
        _tc_body,
        grid_spec=grid_spec,
        out_shape=[jax.ShapeDtypeStruct((_Q,), jnp.float32),
                   jax.ShapeDtypeStruct((_Q,), jnp.int32)],
    )(x, observations)


def _merge_body(minv_ref, mini_ref, tcv_ref, tci_ref, act_ref,
                out_ref, vout_ref, sem):
    mv = minv_ref[...]
    mi = mini_ref[...]
    imax = jnp.int32(2**31 - 1)
    for q in range(_Q):
        blk = mv[:, q * 16:(q + 1) * 16]
        blki = mi[:, q * 16:(q + 1) * 16]
        m_sc = jnp.min(blk)
        i_sc = jnp.min(jnp.where(blk == m_sc, blki, imax))
        m_tc = tcv_ref[q]
        i_tc = tci_ref[q]
        take_tc = m_tc <= m_sc
        m = jnp.where(take_tc, m_tc, m_sc)
        bi = jnp.where(take_tc, i_tc, i_sc)
        copy = pltpu.make_async_copy(
            act_ref.at[pl.ds(bi, 1)], vout_ref.at[pl.ds(q, 1)], sem)
        copy.start()
        copy.wait()
        row = vout_ref[pl.ds(q, 1), :]
        out_ref[pl.ds(q, 1), :] = jnp.where(
            m < _MIN_DIST, row, jnp.zeros_like(row))


def _merge(minv, mini, tcv, tci, actions):
    return pl.pallas_call(
        _merge_body,
        in_specs=[
            pl.BlockSpec((_NW, _Q * 16), lambda: (0, 0)),
            pl.BlockSpec((_NW, _Q * 16), lambda: (0, 0)),
            pl.BlockSpec(memory_space=pltpu.SMEM),
            pl.BlockSpec(memory_space=pltpu.SMEM),
            pl.BlockSpec(memory_space=pl.ANY),
        ],
        out_specs=pl.BlockSpec((_Q, _OUT), lambda: (0, 0)),
        scratch_shapes=[
            pltpu.VMEM((_Q, _OUT), jnp.float32),
            pltpu.SemaphoreType.DMA,
        ],
        out_shape=jax.ShapeDtypeStruct((_Q, _OUT), jnp.float32),
    )(minv, mini, tcv, tci, actions)


@jax.jit
def kernel(x, observations, actions):
    deltas = jnp.concatenate([x[:1], x[:-1] - x[1:]], axis=0)
    dflat = deltas.reshape(-1)
    dsplat = jnp.broadcast_to(dflat[:, None], (_Q * _D, 16))
    dasplat = jnp.broadcast_to(jnp.abs(dflat)[:, None], (_Q * _D, 16))
    obsT_sc = observations[_KTC:].T
    minv, mini = _sc_scan(dsplat, dasplat, obsT_sc)
    tcv, tci = _tc_scan(x, observations)
    return _merge(minv, mini, tcv, tci, actions)


# speedup vs baseline: 2.5434x; 2.5434x over previous
import functools
import jax
import jax.numpy as jnp
from jax import lax
from jax.experimental import pallas as pl
from jax.experimental.pallas import tpu as pltpu
from jax.experimental.pallas import tpu_sc as plsc

_Q, _K, _D, _OUT = 16, 100000, 128, 18
_MIN_DIST = float(_OUT)

_BLK = 2000
_KTC = 50000
_NBLK = _KTC // _BLK
_KSC = _K - _KTC
_NW = 32
_RPW = _KSC // _NW
_NG = (_RPW // 16 + 3) // 2 * 2


def _sc_body(dspl_hbm, daspl_hbm, obsT_hbm, outv_hbm, outi_hbm,
             dtab, datab, buf0, buf1, minbuf, idxbuf,
             sem0, sem1, semt0, semt1):
    c = lax.axis_index("c")
    s = lax.axis_index("s")
    w = s * 2 + c
    a0 = (w * _RPW) // 16 * 16
    a1 = jnp.where(w == _NW - 1, _KSC, ((w + 1) * _RPW) // 16 * 16)
    last = a1 - 16

    cpt0 = pltpu.make_async_copy(dspl_hbm, dtab, semt0)
    cpt1 = pltpu.make_async_copy(daspl_hbm, datab, semt1)
    cpt0.start()
    cpt1.start()

    for q in range(_Q):
        minbuf[pl.ds(q * 16, 16)] = jnp.full((16,), jnp.inf, jnp.float32)
        idxbuf[pl.ds(q * 16, 16)] = jnp.zeros((16,), jnp.int32)

    lane16 = lax.broadcasted_iota(jnp.int32, (16,), 0)

    def gstart(g):
        return jnp.minimum(a0 + 16 * g, last)

    def dma(g, buf, sem):
        return pltpu.make_async_copy(
            obsT_hbm.at[:, pl.ds(gstart(g), 16)], buf, sem)

    dma(0, buf0, sem0).start()
    dma(1, buf1, sem1).start()
    cpt0.wait()
    cpt1.wait()

    def group(g, buf, sem):
        dma(g, buf, sem).wait()
        rowvec = lane16 + (gstart(g) + _KTC)

        def dstep(d, accs):
            ovec = buf[d]
            oabs = jnp.abs(ovec)
            out = []
            for q in range(_Q):
                dq = dtab[_D * q + d]
                daq = datab[_D * q + d]
                num = jnp.abs(ovec - dq)
                den = jnp.maximum(oabs + daq, 1e-30)
                out.append(accs[q] + num / den)
            return tuple(out)

        zero = jnp.zeros((16,), jnp.float32)
        accs = lax.fori_loop(0, _D, dstep, tuple(zero for _ in range(_Q)))

        for q in range(_Q):
            sl = pl.ds(q * 16, 16)
            mv = minbuf[sl]
            better = accs[q] < mv
            minbuf[sl] = jnp.where(better, accs[q], mv)
            idxbuf[sl] = jnp.where(better, rowvec, idxbuf[sl])

        @pl.when(g + 2 < _NG)
        def _pref():
            dma(g + 2, buf, sem).start()

    def outer(o, carry):
        group(o * 2, buf0, sem0)
        group(o * 2 + 1, buf1, sem1)
        return carry

    lax.fori_loop(0, _NG // 2, outer, jnp.int32(0))

    pltpu.sync_copy(minbuf, outv_hbm.at[w])
    pltpu.sync_copy(idxbuf, outi_hbm.at[w])


def _sc_scan(dsplat, dasplat, obsT):
    mesh = plsc.VectorSubcoreMesh(core_axis_name="c", subcore_axis_name="s")
    f = pl.kernel(
        _sc_body,
        out_type=[jax.ShapeDtypeStruct((_NW, _Q * 16), jnp.float32),
                  jax.ShapeDtypeStruct((_NW, _Q * 16), jnp.int32)],
        mesh=mesh,
        scratch_types=[
            pltpu.VMEM((_Q * _D, 16), jnp.float32),
            pltpu.VMEM((_Q * _D, 16), jnp.float32),
            pltpu.VMEM((_D, 16), jnp.float32),
            pltpu.VMEM((_D, 16), jnp.float32),
            pltpu.VMEM((_Q * 16,), jnp.float32),
            pltpu.VMEM((_Q * 16,), jnp.int32),
            pltpu.SemaphoreType.DMA,
            pltpu.SemaphoreType.DMA,
            pltpu.SemaphoreType.DMA,
            pltpu.SemaphoreType.DMA,
        ],
        compiler_params=pltpu.CompilerParams(use_tc_tiling_on_sc=False),
    )
    return f(dsplat, dasplat, obsT)


def _tc_body(x_ref, obs_ref, outv_ref, outi_ref):
    b = pl.program_id(0)

    @pl.when(b == 0)
    def _init():
        for q in range(_Q):
            outv_ref[q] = jnp.inf
            outi_ref[q] = 0

    x = x_ref[...]
    deltas = jnp.concatenate([x[:1], x[:-1] - x[1:]], axis=0)
    dabs = jnp.abs(deltas)

    obs = obs_ref[...]
    oabs = jnp.abs(obs)
    base = b * _BLK

    for q in range(_Q):
        num = jnp.abs(obs - deltas[q][None, :])
        den = jnp.maximum(oabs + dabs[q][None, :], 1e-30)
        dist = jnp.sum(num / den, axis=1)
        bmin = jnp.min(dist)

        @pl.when(bmin < outv_ref[q])
        def _upd():
            rows = jax.lax.broadcasted_iota(jnp.int32, (_BLK,), 0) + base
            bidx = jnp.min(
                jnp.where(dist == bmin, rows, jnp.int32(2**31 - 1))
            )
            outv_ref[q] = bmin
            outi_ref[q] = bidx


def _tc_scan(x, observations):
    grid_spec = pltpu.PrefetchScalarGridSpec(
        num_scalar_prefetch=0,
        grid=(_NBLK,),
        in_specs=[
            pl.BlockSpec((_Q, _D), lambda b: (0, 0)),
            pl.BlockSpec((_BLK, _D), lambda b: (b, 0)),
        ],
        out_specs=[
            pl.BlockSpec(memory_space=pltpu.SMEM),
            pl.BlockSpec(memory_space=pltpu.SMEM),
        ],
    )
    return pl.pallas_call(
        _tc_body,
        grid_spec=grid_spec,
        out_shape=[jax.ShapeDtypeStruct((_Q,), jnp.float32),
                   jax.ShapeDtypeStruct((_Q,), jnp.int32)],
    )(x, observations)


def _merge_body(minv_ref, mini_ref, tcv_ref, tci_ref, act_ref,
                out_ref, vout_ref, sem):
    mv = minv_ref[...]
    mi = mini_ref[...]
    imax = jnp.int32(2**31 - 1)
    for q in range(_Q):
        blk = mv[:, q * 16:(q + 1) * 16]
        blki = mi[:, q * 16:(q + 1) * 16]
        m_sc = jnp.min(blk)
        i_sc = jnp.min(jnp.where(blk == m_sc, blki, imax))
        m_tc = tcv_ref[q]
        i_tc = tci_ref[q]
        take_tc = m_tc <= m_sc
        m = jnp.where(take_tc, m_tc, m_sc)
        bi = jnp.where(take_tc, i_tc, i_sc)
        copy = pltpu.make_async_copy(
            act_ref.at[pl.ds(bi, 1)], vout_ref.at[pl.ds(q, 1)], sem)
        copy.start()
        copy.wait()
        row = vout_ref[pl.ds(q, 1), :]
        out_ref[pl.ds(q, 1), :] = jnp.where(
            m < _MIN_DIST, row, jnp.zeros_like(row))


def _merge(minv, mini, tcv, tci, actions):
    return pl.pallas_call(
        _merge_body,
        in_specs=[
            pl.BlockSpec((_NW, _Q * 16), lambda: (0, 0)),
            pl.BlockSpec((_NW, _Q * 16), lambda: (0, 0)),
            pl.BlockSpec(memory_space=pltpu.SMEM),
            pl.BlockSpec(memory_space=pltpu.SMEM),
            pl.BlockSpec(memory_space=pl.ANY),
        ],
        out_specs=pl.BlockSpec((_Q, _OUT), lambda: (0, 0)),
        scratch_shapes=[
            pltpu.VMEM((_Q, _OUT), jnp.float32),
            pltpu.SemaphoreType.DMA,
        ],
        out_shape=jax.ShapeDtypeStruct((_Q, _OUT), jnp.float32),
    )(minv, mini, tcv, tci, actions)


@jax.jit
def kernel(x, observations, actions):
    deltas = jnp.concatenate([x[:1], x[:-1] - x[1:]], axis=0)
    dflat = deltas.reshape(-1)
    dsplat = jnp.broadcast_to(dflat[:, None], (_Q * _D, 16))
    dasplat = jnp.broadcast_to(jnp.abs(dflat)[:, None], (_Q * _D, 16))
    obsT_sc = observations[_KTC:].T
    minv, mini = _sc_scan(dsplat, dasplat, obsT_sc)
    tcv, tci = _tc_scan(x, observations)
    return _merge(minv, mini, tcv, tci, actions)


# speedup vs baseline: 2.8099x; 1.1048x over previous
import functools
import jax
import jax.numpy as jnp
from jax import lax
from jax.experimental import pallas as pl
from jax.experimental.pallas import tpu as pltpu
from jax.experimental.pallas import tpu_sc as plsc

_Q, _K, _D, _OUT = 16, 100000, 128, 18
_MIN_DIST = float(_OUT)

_BLK = 2000
_KTC = 52000
_NBLK = _KTC // _BLK
_KSC = _K - _KTC
_NW = 32
_RPW = _KSC // _NW
_NG = (_RPW // 16 + 3) // 2 * 2


def _sc_body(dspl_hbm, daspl_hbm, obsT_hbm, outv_hbm, outi_hbm,
             dtab, datab, buf0, buf1, minbuf, idxbuf,
             sem0, sem1, semt0, semt1):
    c = lax.axis_index("c")
    s = lax.axis_index("s")
    w = s * 2 + c
    a0 = (w * _RPW) // 16 * 16
    a1 = jnp.where(w == _NW - 1, _KSC, ((w + 1) * _RPW) // 16 * 16)
    last = a1 - 16

    cpt0 = pltpu.make_async_copy(dspl_hbm, dtab, semt0)
    cpt1 = pltpu.make_async_copy(daspl_hbm, datab, semt1)
    cpt0.start()
    cpt1.start()

    for q in range(_Q):
        minbuf[pl.ds(q * 16, 16)] = jnp.full((16,), jnp.inf, jnp.float32)
        idxbuf[pl.ds(q * 16, 16)] = jnp.zeros((16,), jnp.int32)

    lane16 = lax.broadcasted_iota(jnp.int32, (16,), 0)

    def gstart(g):
        return jnp.minimum(a0 + 16 * g, last)

    def dma(g, buf, sem):
        return pltpu.make_async_copy(
            obsT_hbm.at[:, pl.ds(gstart(g), 16)], buf, sem)

    dma(0, buf0, sem0).start()
    dma(1, buf1, sem1).start()
    cpt0.wait()
    cpt1.wait()

    def group(g, buf, sem):
        dma(g, buf, sem).wait()
        rowvec = lane16 + (gstart(g) + _KTC)

        def dstep(d, accs):
            ovec = buf[d]
            oabs = jnp.abs(ovec)
            out = []
            for q in range(_Q):
                dq = dtab[_D * q + d]
                daq = datab[_D * q + d]
                num = jnp.abs(ovec - dq)
                den = jnp.maximum(oabs + daq, 1e-30)
                out.append(accs[q] + num / den)
            return tuple(out)

        zero = jnp.zeros((16,), jnp.float32)
        accs = lax.fori_loop(0, _D, dstep, tuple(zero for _ in range(_Q)))

        for q in range(_Q):
            sl = pl.ds(q * 16, 16)
            mv = minbuf[sl]
            better = accs[q] < mv
            minbuf[sl] = jnp.where(better, accs[q], mv)
            idxbuf[sl] = jnp.where(better, rowvec, idxbuf[sl])

        @pl.when(g + 2 < _NG)
        def _pref():
            dma(g + 2, buf, sem).start()

    def outer(o, carry):
        group(o * 2, buf0, sem0)
        group(o * 2 + 1, buf1, sem1)
        return carry

    lax.fori_loop(0, _NG // 2, outer, jnp.int32(0))

    pltpu.sync_copy(minbuf, outv_hbm.at[w])
    pltpu.sync_copy(idxbuf, outi_hbm.at[w])


def _sc_scan(dsplat, dasplat, obsT):
    mesh = plsc.VectorSubcoreMesh(core_axis_name="c", subcore_axis_name="s")
    f = pl.kernel(
        _sc_body,
        out_type=[jax.ShapeDtypeStruct((_NW, _Q * 16), jnp.float32),
                  jax.ShapeDtypeStruct((_NW, _Q * 16), jnp.int32)],
        mesh=mesh,
        scratch_types=[
            pltpu.VMEM((_Q * _D, 16), jnp.float32),
            pltpu.VMEM((_Q * _D, 16), jnp.float32),
            pltpu.VMEM((_D, 16), jnp.float32),
            pltpu.VMEM((_D, 16), jnp.float32),
            pltpu.VMEM((_Q * 16,), jnp.float32),
            pltpu.VMEM((_Q * 16,), jnp.int32),
            pltpu.SemaphoreType.DMA,
            pltpu.SemaphoreType.DMA,
            pltpu.SemaphoreType.DMA,
            pltpu.SemaphoreType.DMA,
        ],
        compiler_params=pltpu.CompilerParams(use_tc_tiling_on_sc=False),
    )
    return f(dsplat, dasplat, obsT)


def _tc_body(x_ref, obs_ref, outv_ref, outi_ref):
    b = pl.program_id(0)

    @pl.when(b == 0)
    def _init():
        for q in range(_Q):
            outv_ref[q] = jnp.inf
            outi_ref[q] = 0

    x = x_ref[...]
    deltas = jnp.concatenate([x[:1], x[:-1] - x[1:]], axis=0)
    dabs = jnp.abs(deltas)

    obs = obs_ref[...]
    oabs = jnp.abs(obs)
    base = b * _BLK

    for q in range(_Q):
        num = jnp.abs(obs - deltas[q][None, :])
        den = jnp.maximum(oabs + dabs[q][None, :], 1e-30)
        dist = jnp.sum(num / den, axis=1)
        bmin = jnp.min(dist)

        @pl.when(bmin < outv_ref[q])
        def _upd():
            rows = jax.lax.broadcasted_iota(jnp.int32, (_BLK,), 0) + base
            bidx = jnp.min(
                jnp.where(dist == bmin, rows, jnp.int32(2**31 - 1))
            )
            outv_ref[q] = bmin
            outi_ref[q] = bidx


def _tc_scan(x, observations):
    grid_spec = pltpu.PrefetchScalarGridSpec(
        num_scalar_prefetch=0,
        grid=(_NBLK,),
        in_specs=[
            pl.BlockSpec((_Q, _D), lambda b: (0, 0)),
            pl.BlockSpec((_BLK, _D), lambda b: (b, 0)),
        ],
        out_specs=[
            pl.BlockSpec(memory_space=pltpu.SMEM),
            pl.BlockSpec(memory_space=pltpu.SMEM),
        ],
    )
    return pl.pallas_call(
        _tc_body,
        grid_spec=grid_spec,
        out_shape=[jax.ShapeDtypeStruct((_Q,), jnp.float32),
                   jax.ShapeDtypeStruct((_Q,), jnp.int32)],
    )(x, observations)


def _merge_body(minv_ref, mini_ref, tcv_ref, tci_ref, act_ref,
                out_ref, vout_ref, sem):
    mv = minv_ref[...]
    mi = mini_ref[...]
    imax = jnp.int32(2**31 - 1)
    for q in range(_Q):
        blk = mv[:, q * 16:(q + 1) * 16]
        blki = mi[:, q * 16:(q + 1) * 16]
        m_sc = jnp.min(blk)
        i_sc = jnp.min(jnp.where(blk == m_sc, blki, imax))
        m_tc = tcv_ref[q]
        i_tc = tci_ref[q]
        take_tc = m_tc <= m_sc
        m = jnp.where(take_tc, m_tc, m_sc)
        bi = jnp.where(take_tc, i_tc, i_sc)
        copy = pltpu.make_async_copy(
            act_ref.at[pl.ds(bi, 1)], vout_ref.at[pl.ds(q, 1)], sem)
        copy.start()
        copy.wait()
        row = vout_ref[pl.ds(q, 1), :]
        out_ref[pl.ds(q, 1), :] = jnp.where(
            m < _MIN_DIST, row, jnp.zeros_like(row))


def _merge(minv, mini, tcv, tci, actions):
    return pl.pallas_call(
        _merge_body,
        in_specs=[
            pl.BlockSpec((_NW, _Q * 16), lambda: (0, 0)),
            pl.BlockSpec((_NW, _Q * 16), lambda: (0, 0)),
            pl.BlockSpec(memory_space=pltpu.SMEM),
            pl.BlockSpec(memory_space=pltpu.SMEM),
            pl.BlockSpec(memory_space=pl.ANY),
        ],
        out_specs=pl.BlockSpec((_Q, _OUT), lambda: (0, 0)),
        scratch_shapes=[
            pltpu.VMEM((_Q, _OUT), jnp.float32),
            pltpu.SemaphoreType.DMA,
        ],
        out_shape=jax.ShapeDtypeStruct((_Q, _OUT), jnp.float32),
    )(minv, mini, tcv, tci, actions)


@jax.jit
def kernel(x, observations, actions):
    deltas = jnp.concatenate([x[:1], x[:-1] - x[1:]], axis=0)
    dflat = deltas.reshape(-1)
    dsplat = jnp.broadcast_to(dflat[:, None], (_Q * _D, 16))
    dasplat = jnp.broadcast_to(jnp.abs(dflat)[:, None], (_Q * _D, 16))
    obsT_sc = observations[_KTC:].T
    minv, mini = _sc_scan(dsplat, dasplat, obsT_sc)
    tcv, tci = _tc_scan(x, observations)
    return _merge(minv, mini, tcv, tci, actions)
